# baseline (speedup 1.0000x reference)
import jax
import jax.numpy as jnp
from jax.experimental import pallas as pl
from jax.experimental.pallas import tpu as pltpu

_NQ = 8
_K = 1024
_D = 256
_BLK = 2048


def _row_sumsq(x):
    s = x * x
    s = s[:, :128] + s[:, 128:]
    acc = s[:, 0:8]
    for j in range(1, 16):
        acc = acc + s[:, 8 * j:8 * j + 8]
    acc = acc[:, :4] + acc[:, 4:]
    acc = acc[:, :2] + acc[:, 2:]
    return acc[:, :1] + acc[:, 1:]


def _rvq_block_kernel(z_ref, cb_ref, q_ref, idx_ref, loss_ref):
    z = z_ref[...]
    residual = z
    quantized = jnp.zeros_like(z)
    loss_sum = jnp.float32(0.0)
    lane_iota = jax.lax.broadcasted_iota(jnp.int32, (_BLK, _K), 1)
    for i in range(_NQ):
        cb = cb_ref[i]
        rnorm = _row_sumsq(residual)
        cbnorm = _row_sumsq(cb).reshape(1, _K)
        prod = jax.lax.dot_general(
            residual, cb, (((1,), (1,)), ((), ())),
            preferred_element_type=jnp.float32)
        d2 = (rnorm - 2.0 * prod) + cbnorm
        min_d = jnp.min(d2, axis=1, keepdims=True)
        idx = jnp.min(jnp.where(d2 == min_d, lane_iota, _K), axis=1)
        onehot = (lane_iota == idx[:, None]).astype(jnp.float32)
        ql = jnp.dot(onehot, cb, precision=jax.lax.Precision.HIGHEST,
                     preferred_element_type=jnp.float32)
        quantized = quantized + ql
        residual = residual - ql
        loss_sum = loss_sum + jnp.sum(residual * residual)
        idx_ref[i, :] = idx
    q_ref[...] = quantized

    @pl.when(pl.program_id(0) == 0)
    def _init():
        loss_ref[0, 0] = jnp.float32(0.0)

    loss_ref[0, 0] += loss_sum


def kernel(z, codebooks):
    B, T, D = z.shape
    ntok = B * T
    zf = z.reshape(ntok, D)
    nblocks = ntok // _BLK
    q, idx, loss = pl.pallas_call(
        _rvq_block_kernel,
        grid=(nblocks,),
        in_specs=[
            pl.BlockSpec((_BLK, _D), lambda i: (i, 0)),
            pl.BlockSpec((_NQ, _K, _D), lambda i: (0, 0, 0)),
        ],
        out_specs=[
            pl.BlockSpec((_BLK, _D), lambda i: (i, 0)),
            pl.BlockSpec((_NQ, _BLK), lambda i: (0, i)),
            pl.BlockSpec(memory_space=pltpu.SMEM),
        ],
        out_shape=[
            jax.ShapeDtypeStruct((ntok, _D), jnp.float32),
            jax.ShapeDtypeStruct((_NQ, ntok), jnp.int32),
            jax.ShapeDtypeStruct((1, 1), jnp.float32),
        ],
    )(zf, codebooks)
    quantized_st = q.reshape(B, T, D)
    indices = idx.reshape(_NQ, B, T).transpose(1, 0, 2)
    commitment_loss = loss[0, 0] / jnp.float32(_NQ * ntok * _D)
    return quantized_st, indices, commitment_loss


# speedup vs baseline: 2.2161x; 2.2161x over previous
import jax
import jax.numpy as jnp
from jax.experimental import pallas as pl
from jax.experimental.pallas import tpu as pltpu

_NQ = 8
_K = 1024
_D = 256
_BLK = 2048


def _col_sumsq(x):
    s = x * x
    s = s[:128, :] + s[128:, :]
    acc = s[0:8, :]
    for j in range(1, 16):
        acc = acc + s[8 * j:8 * j + 8, :]
    acc = acc[:4, :] + acc[4:, :]
    acc = acc[:2, :] + acc[2:, :]
    return acc[:1, :] + acc[1:, :]


def _rvq_block_kernel(zt_ref, cb_ref, cbt_ref, qt_ref, idx_ref, loss_ref):
    rest = zt_ref[...]
    quant = jnp.zeros_like(rest)
    loss_sum = jnp.float32(0.0)
    sub_iota = jax.lax.broadcasted_iota(jnp.int32, (_K, _BLK), 0)

    cbn_rows = jnp.concatenate(
        [_col_sumsq(cbt_ref[i]) for i in range(_NQ)], axis=0)
    cbn_cols = cbn_rows.T

    for i in range(_NQ):
        rnorm = _col_sumsq(rest)
        prod = jax.lax.dot_general(
            cb_ref[i], rest, (((1,), (0,)), ((), ())),
            preferred_element_type=jnp.float32)
        d2 = (rnorm - 2.0 * prod) + cbn_cols[:, i:i + 1]
        min_d = jnp.min(d2, axis=0, keepdims=True)
        idx = jnp.min(jnp.where(d2 == min_d, sub_iota, _K),
                      axis=0, keepdims=True)
        onehot = (sub_iota == idx).astype(jnp.float32)
        ql = jax.lax.dot_general(
            cbt_ref[i], onehot, (((1,), (0,)), ((), ())),
            precision=jax.lax.Precision.HIGHEST,
            preferred_element_type=jnp.float32)
        quant = quant + ql
        rest = rest - ql
        loss_sum = loss_sum + jnp.sum(rest * rest)
        idx_ref[i, :] = idx[0, :]
    qt_ref[...] = quant

    @pl.when(pl.program_id(0) == 0)
    def _init():
        loss_ref[0, 0] = jnp.float32(0.0)

    loss_ref[0, 0] += loss_sum


def kernel(z, codebooks):
    B, T, D = z.shape
    ntok = B * T
    zt = z.reshape(ntok, D).T
    cbt = codebooks.transpose(0, 2, 1)
    nblocks = ntok // _BLK
    qt, idx, loss = pl.pallas_call(
        _rvq_block_kernel,
        grid=(nblocks,),
        in_specs=[
            pl.BlockSpec((_D, _BLK), lambda i: (0, i)),
            pl.BlockSpec((_NQ, _K, _D), lambda i: (0, 0, 0)),
            pl.BlockSpec((_NQ, _D, _K), lambda i: (0, 0, 0)),
        ],
        out_specs=[
            pl.BlockSpec((_D, _BLK), lambda i: (0, i)),
            pl.BlockSpec((_NQ, _BLK), lambda i: (0, i)),
            pl.BlockSpec(memory_space=pltpu.SMEM),
        ],
        out_shape=[
            jax.ShapeDtypeStruct((_D, ntok), jnp.float32),
            jax.ShapeDtypeStruct((_NQ, ntok), jnp.int32),
            jax.ShapeDtypeStruct((1, 1), jnp.float32),
        ],
    )(zt, codebooks, cbt)
    quantized_st = qt.T.reshape(B, T, D)
    indices = idx.reshape(_NQ, B, T).transpose(1, 0, 2)
    commitment_loss = loss[0, 0] / jnp.float32(_NQ * ntok * _D)
    return quantized_st, indices, commitment_loss


# speedup vs baseline: 2.2214x; 1.0024x over previous
import jax
import jax.numpy as jnp
from jax.experimental import pallas as pl
from jax.experimental.pallas import tpu as pltpu

_NQ = 8
_K = 1024
_D = 256
_BLK = 2048


def _col_sumsq(x):
    s = x * x
    s = s[:128, :] + s[128:, :]
    acc = s[0:8, :]
    for j in range(1, 16):
        acc = acc + s[8 * j:8 * j + 8, :]
    acc = acc[:4, :] + acc[4:, :]
    acc = acc[:2, :] + acc[2:, :]
    return acc[:1, :] + acc[1:, :]


def _rvq_block_kernel(zt_ref, cb_ref, cbt_ref, qt_ref, idx_ref, loss_ref):
    rest = zt_ref[...]
    quant = jnp.zeros_like(rest)
    loss_sum = jnp.float32(0.0)
    sub_iota = jax.lax.broadcasted_iota(jnp.int32, (_K, _BLK), 0)

    cbn_rows = jnp.concatenate(
        [_col_sumsq(cbt_ref[i]) for i in range(_NQ)], axis=0)
    cbn_cols = cbn_rows.T

    for i in range(_NQ):
        rnorm = _col_sumsq(rest)
        prod = jax.lax.dot_general(
            cb_ref[i], rest, (((1,), (0,)), ((), ())),
            preferred_element_type=jnp.float32)
        d2 = (rnorm - 2.0 * prod) + cbn_cols[:, i:i + 1]
        min_d = jnp.min(d2, axis=0, keepdims=True)
        idx = jnp.min(jnp.where(d2 == min_d, sub_iota, _K),
                      axis=0, keepdims=True)
        onehot = (sub_iota == idx).astype(jnp.float32)
        ql = jax.lax.dot_general(
            cbt_ref[i], onehot, (((1,), (0,)), ((), ())),
            precision=jax.lax.Precision.HIGHEST,
            preferred_element_type=jnp.float32)
        quant = quant + ql
        rest = rest - ql
        loss_sum = loss_sum + jnp.sum(rest * rest)
        idx_ref[i, :] = idx[0, :]
    qt_ref[...] = quant
    loss_ref[0, 0, 0] = loss_sum


def kernel(z, codebooks):
    B, T, D = z.shape
    ntok = B * T
    zt = z.reshape(ntok, D).T
    cbt = codebooks.transpose(0, 2, 1)
    nblocks = ntok // _BLK
    qt, idx, loss = pl.pallas_call(
        _rvq_block_kernel,
        grid=(nblocks,),
        in_specs=[
            pl.BlockSpec((_D, _BLK), lambda i: (0, i)),
            pl.BlockSpec((_NQ, _K, _D), lambda i: (0, 0, 0)),
            pl.BlockSpec((_NQ, _D, _K), lambda i: (0, 0, 0)),
        ],
        out_specs=[
            pl.BlockSpec((_D, _BLK), lambda i: (0, i)),
            pl.BlockSpec((_NQ, _BLK), lambda i: (0, i)),
            pl.BlockSpec((1, 1, 1), lambda i: (i, 0, 0),
                         memory_space=pltpu.SMEM),
        ],
        out_shape=[
            jax.ShapeDtypeStruct((_D, ntok), jnp.float32),
            jax.ShapeDtypeStruct((_NQ, ntok), jnp.int32),
            jax.ShapeDtypeStruct((nblocks, 1, 1), jnp.float32),
        ],
        compiler_params=pltpu.CompilerParams(
            dimension_semantics=("parallel",)),
    )(zt, codebooks, cbt)
    quantized_st = qt.T.reshape(B, T, D)
    indices = idx.reshape(_NQ, B, T).transpose(1, 0, 2)
    commitment_loss = jnp.sum(loss) / jnp.float32(_NQ * ntok * _D)
    return quantized_st, indices, commitment_loss
